# Initial kernel scaffold; baseline (speedup 1.0000x reference)
import jax
import jax.numpy as jnp
from jax.experimental import pallas as pl


def kernel(word_embeddings, pos_table):
    raise NotImplementedError("write your pallas kernel here")


# speedup vs baseline: 2.8271x; 2.8271x over previous
import jax
import jax.numpy as jnp
from jax.experimental import pallas as pl


def _add_kernel(we_ref, pos_ref, out_ref):
    out_ref[...] = we_ref[...] + pos_ref[...]


def kernel(word_embeddings, pos_table):
    B, S, D = word_embeddings.shape
    BS = 512
    grid = (S // BS, B)
    return pl.pallas_call(
        _add_kernel,
        grid=grid,
        in_specs=[
            pl.BlockSpec((1, BS, D), lambda s, b: (b, s, 0)),
            pl.BlockSpec((BS, D), lambda s, b: (s, 0)),
        ],
        out_specs=pl.BlockSpec((1, BS, D), lambda s, b: (b, s, 0)),
        out_shape=jax.ShapeDtypeStruct((B, S, D), word_embeddings.dtype),
    )(word_embeddings, pos_table)


# speedup vs baseline: 3.2841x; 1.1616x over previous
import jax
import jax.numpy as jnp
from jax.experimental import pallas as pl


def _add_kernel(we_ref, pos_ref, out_ref):
    out_ref[...] = we_ref[...] + pos_ref[...][None, :, :]


def kernel(word_embeddings, pos_table):
    B, S, D = word_embeddings.shape
    BS = 512
    grid = (S // BS,)
    return pl.pallas_call(
        _add_kernel,
        grid=grid,
        in_specs=[
            pl.BlockSpec((B, BS, D), lambda s: (0, s, 0)),
            pl.BlockSpec((BS, D), lambda s: (s, 0)),
        ],
        out_specs=pl.BlockSpec((B, BS, D), lambda s: (0, s, 0)),
        out_shape=jax.ShapeDtypeStruct((B, S, D), word_embeddings.dtype),
    )(word_embeddings, pos_table)
